# baseline (speedup 1.0000x reference)
import jax
import jax.numpy as jnp
from jax import lax
from jax.experimental import pallas as pl
from jax.experimental.pallas import tpu as pltpu
from jax.experimental.pallas import tpu_sc as plsc

N = 10000
E = 320000
D = 128

NC = 2
NS = 16
NW = NC * NS
EW = E // NW
C = 80
KCH = EW // C
NPAD = 10240
ROWS_PER_TILE = NPAD // NS
NBUF = 4
NIDX = NBUF + 2


def _sc_agg_body(x_hbm, ei_hbm, zero_hbm, out_hbm,
                 srcv, dstv, rows, gsem, ssem, sisem, disem, aggs):
    c = lax.axis_index("c")
    s = lax.axis_index("s")
    w = c * NS + s

    pltpu.sync_copy(zero_hbm,
                    aggs.at[pl.ds(s * ROWS_PER_TILE, ROWS_PER_TILE)])

    def gwait(b):
        pltpu.make_async_copy(x_hbm.at[pl.ds(0, C)], rows.at[b],
                              gsem.at[b]).wait()

    def swait(b):
        pltpu.make_async_copy(x_hbm.at[pl.ds(0, C)], rows.at[b],
                              ssem.at[b]).wait()

    def siwait(j):
        pltpu.make_async_copy(ei_hbm.at[pl.ds(0, C)], srcv.at[j],
                              sisem.at[j]).wait()

    def diwait(j):
        pltpu.make_async_copy(ei_hbm.at[pl.ds(0, C)], dstv.at[j],
                              disem.at[j]).wait()

    def iissue(k, j):
        pltpu.async_copy(ei_hbm.at[pl.ds(w * EW + k * C, C)], srcv.at[j],
                         sisem.at[j])
        pltpu.async_copy(ei_hbm.at[pl.ds(E + w * EW + k * C, C)], dstv.at[j],
                         disem.at[j])

    def gissue(k, j, b):
        pltpu.async_copy(x_hbm.at[srcv.at[j]], rows.at[b], gsem.at[b])

    plsc.subcore_barrier()

    for j in range(NBUF):
        iissue(j, j)
    for j in range(NBUF - 2):
        siwait(j)
        gissue(j, j, j)

    def chunk(k, _):
        b = lax.rem(k, NBUF)
        ib = lax.rem(k, NIDX)
        gwait(b)
        diwait(ib)
        pltpu.async_copy(rows.at[b], aggs.at[dstv.at[ib]], ssem.at[b],
                         add=True)

        @pl.when(k >= 2)
        def _():
            swait(lax.rem(k - 2, NBUF))

        nj = k + NBUF

        @pl.when(nj < KCH)
        def _():
            iissue(nj, lax.rem(nj, NIDX))

        nk = k + NBUF - 2

        @pl.when(nk < KCH)
        def _():
            nib = lax.rem(nk, NIDX)
            siwait(nib)
            gissue(nk, nib, lax.rem(nk, NBUF))
        return 0
    lax.fori_loop(0, KCH, chunk, 0)

    swait((KCH - 2) % NBUF)
    swait((KCH - 1) % NBUF)

    plsc.subcore_barrier()

    pltpu.sync_copy(aggs.at[pl.ds(s * ROWS_PER_TILE, ROWS_PER_TILE)],
                    out_hbm.at[c, pl.ds(s * ROWS_PER_TILE, ROWS_PER_TILE)])


_sc_agg = pl.kernel(
    _sc_agg_body,
    out_type=jax.ShapeDtypeStruct((NC, NPAD, D), jnp.float32),
    mesh=plsc.VectorSubcoreMesh(core_axis_name="c", subcore_axis_name="s"),
    scratch_types=[
        pltpu.VMEM((NIDX, C), jnp.int32),
        pltpu.VMEM((NIDX, C), jnp.int32),
        pltpu.VMEM((NBUF, C, D), jnp.float32),
        pltpu.SemaphoreType.DMA((NBUF,)),
        pltpu.SemaphoreType.DMA((NBUF,)),
        pltpu.SemaphoreType.DMA((NIDX,)),
        pltpu.SemaphoreType.DMA((NIDX,)),
        pltpu.VMEM_SHARED((NPAD, D), jnp.float32),
    ],
)


def _gru_body(aggp_ref, h_ref, wmsg, wz, wr, wh, bz, br, bh, out_ref):
    a = aggp_ref[0] + aggp_ref[1]
    hh = h_ref[...]
    m = jnp.dot(a, wmsg[...], preferred_element_type=jnp.float32)
    z = jax.nn.sigmoid(
        jnp.dot(m, wz[0:D], preferred_element_type=jnp.float32)
        + jnp.dot(hh, wz[D:2 * D], preferred_element_type=jnp.float32)
        + bz[...].reshape(1, D))
    r = jax.nn.sigmoid(
        jnp.dot(m, wr[0:D], preferred_element_type=jnp.float32)
        + jnp.dot(hh, wr[D:2 * D], preferred_element_type=jnp.float32)
        + br[...].reshape(1, D))
    cand = jnp.tanh(
        jnp.dot(m, wh[0:D], preferred_element_type=jnp.float32)
        + jnp.dot(r * hh, wh[D:2 * D], preferred_element_type=jnp.float32)
        + bh[...].reshape(1, D))
    out_ref[...] = (1.0 - z) * hh + z * cand


ROW_BLK = 2000


def _gru_tc(partials, h, W_msg, W_z, W_r, W_h, b_z, b_r, b_h):
    grid = (N // ROW_BLK,)
    full = lambda i: (0, 0)
    return pl.pallas_call(
        _gru_body,
        grid=grid,
        in_specs=[
            pl.BlockSpec((NC, ROW_BLK, D), lambda i: (0, i, 0)),
            pl.BlockSpec((ROW_BLK, D), lambda i: (i, 0)),
            pl.BlockSpec((D, D), full),
            pl.BlockSpec((2 * D, D), full),
            pl.BlockSpec((2 * D, D), full),
            pl.BlockSpec((2 * D, D), full),
            pl.BlockSpec((D,), lambda i: (0,)),
            pl.BlockSpec((D,), lambda i: (0,)),
            pl.BlockSpec((D,), lambda i: (0,)),
        ],
        out_specs=pl.BlockSpec((ROW_BLK, D), lambda i: (i, 0)),
        out_shape=jax.ShapeDtypeStruct((N, D), jnp.float32),
    )(partials, h, W_msg, W_z, W_r, W_h, b_z, b_r, b_h)


@jax.jit
def kernel(x, h, edge_index, W_msg, W_z, b_z, W_r, b_r, W_h, b_h):
    ei = edge_index.astype(jnp.int32).reshape(2 * E)
    zeros = jnp.zeros((ROWS_PER_TILE, D), jnp.float32)
    partials = _sc_agg(x, ei, zeros)
    return _gru_tc(partials, h, W_msg, W_z, W_r, W_h, b_z, b_r, b_h)


# speedup vs baseline: 1.1337x; 1.1337x over previous
import jax
import jax.numpy as jnp
from jax import lax
from jax.experimental import pallas as pl
from jax.experimental.pallas import tpu as pltpu
from jax.experimental.pallas import tpu_sc as plsc

N = 10000
E = 320000
D = 128

NC = 2
NS = 16
NW = NC * NS
EW = E // NW
C = 80
KCH = EW // C
NPAD = 10240
ROWS_PER_TILE = NPAD // NS
NBUF = 4
NIDX = NBUF + 1


def _sc_agg_body(x_hbm, ei_hbm, zero_hbm, out_hbm,
                 srcv, dstv, rows, gsem, ssem, sisem, disem, aggs):
    c = lax.axis_index("c")
    s = lax.axis_index("s")
    w = c * NS + s

    pltpu.sync_copy(zero_hbm,
                    aggs.at[pl.ds(s * ROWS_PER_TILE, ROWS_PER_TILE)])

    def gwait(b):
        pltpu.make_async_copy(x_hbm.at[pl.ds(0, C)], rows.at[b],
                              gsem.at[b]).wait()

    def swait(b):
        pltpu.make_async_copy(x_hbm.at[pl.ds(0, C)], rows.at[b],
                              ssem.at[b]).wait()

    def siwait(j):
        pltpu.make_async_copy(ei_hbm.at[pl.ds(0, C)], srcv.at[j],
                              sisem.at[j]).wait()

    def diwait(j):
        pltpu.make_async_copy(ei_hbm.at[pl.ds(0, C)], dstv.at[j],
                              disem.at[j]).wait()

    def iissue(k, j):
        pltpu.async_copy(ei_hbm.at[pl.ds(w * EW + k * C, C)], srcv.at[j],
                         sisem.at[j])
        pltpu.async_copy(ei_hbm.at[pl.ds(E + w * EW + k * C, C)], dstv.at[j],
                         disem.at[j])

    def gissue(k, j, b):
        pltpu.async_copy(x_hbm.at[srcv.at[j]], rows.at[b], gsem.at[b])

    plsc.subcore_barrier()

    for j in range(NBUF):
        iissue(j, j)
    for j in range(NBUF - 1):
        siwait(j)
        gissue(j, j, j)

    def chunk(k, _):
        b = lax.rem(k, NBUF)
        ib = lax.rem(k, NIDX)
        gwait(b)
        diwait(ib)
        pltpu.async_copy(rows.at[b], aggs.at[dstv.at[ib]], ssem.at[b],
                         add=True)

        @pl.when(k >= 1)
        def _():
            swait(lax.rem(k - 1, NBUF))

        nj = k + NBUF

        @pl.when(nj < KCH)
        def _():
            iissue(nj, lax.rem(nj, NIDX))

        nk = k + NBUF - 1

        @pl.when(nk < KCH)
        def _():
            nib = lax.rem(nk, NIDX)
            siwait(nib)
            gissue(nk, nib, lax.rem(nk, NBUF))
        return 0
    lax.fori_loop(0, KCH, chunk, 0)

    swait((KCH - 1) % NBUF)

    plsc.subcore_barrier()

    pltpu.sync_copy(aggs.at[pl.ds(s * ROWS_PER_TILE, ROWS_PER_TILE)],
                    out_hbm.at[c, pl.ds(s * ROWS_PER_TILE, ROWS_PER_TILE)])


_sc_agg = pl.kernel(
    _sc_agg_body,
    out_type=jax.ShapeDtypeStruct((NC, NPAD, D), jnp.float32),
    mesh=plsc.VectorSubcoreMesh(core_axis_name="c", subcore_axis_name="s"),
    scratch_types=[
        pltpu.VMEM((NIDX, C), jnp.int32),
        pltpu.VMEM((NIDX, C), jnp.int32),
        pltpu.VMEM((NBUF, C, D), jnp.float32),
        pltpu.SemaphoreType.DMA((NBUF,)),
        pltpu.SemaphoreType.DMA((NBUF,)),
        pltpu.SemaphoreType.DMA((NIDX,)),
        pltpu.SemaphoreType.DMA((NIDX,)),
        pltpu.VMEM_SHARED((NPAD, D), jnp.float32),
    ],
)


def _gru_body(aggp_ref, h_ref, wmsg, wz, wr, wh, bz, br, bh, out_ref):
    a = aggp_ref[0] + aggp_ref[1]
    hh = h_ref[...]
    m = jnp.dot(a, wmsg[...], preferred_element_type=jnp.float32)
    z = jax.nn.sigmoid(
        jnp.dot(m, wz[0:D], preferred_element_type=jnp.float32)
        + jnp.dot(hh, wz[D:2 * D], preferred_element_type=jnp.float32)
        + bz[...].reshape(1, D))
    r = jax.nn.sigmoid(
        jnp.dot(m, wr[0:D], preferred_element_type=jnp.float32)
        + jnp.dot(hh, wr[D:2 * D], preferred_element_type=jnp.float32)
        + br[...].reshape(1, D))
    cand = jnp.tanh(
        jnp.dot(m, wh[0:D], preferred_element_type=jnp.float32)
        + jnp.dot(r * hh, wh[D:2 * D], preferred_element_type=jnp.float32)
        + bh[...].reshape(1, D))
    out_ref[...] = (1.0 - z) * hh + z * cand


ROW_BLK = 2000


def _gru_tc(partials, h, W_msg, W_z, W_r, W_h, b_z, b_r, b_h):
    grid = (N // ROW_BLK,)
    full = lambda i: (0, 0)
    return pl.pallas_call(
        _gru_body,
        grid=grid,
        in_specs=[
            pl.BlockSpec((NC, ROW_BLK, D), lambda i: (0, i, 0)),
            pl.BlockSpec((ROW_BLK, D), lambda i: (i, 0)),
            pl.BlockSpec((D, D), full),
            pl.BlockSpec((2 * D, D), full),
            pl.BlockSpec((2 * D, D), full),
            pl.BlockSpec((2 * D, D), full),
            pl.BlockSpec((D,), lambda i: (0,)),
            pl.BlockSpec((D,), lambda i: (0,)),
            pl.BlockSpec((D,), lambda i: (0,)),
        ],
        out_specs=pl.BlockSpec((ROW_BLK, D), lambda i: (i, 0)),
        out_shape=jax.ShapeDtypeStruct((N, D), jnp.float32),
    )(partials, h, W_msg, W_z, W_r, W_h, b_z, b_r, b_h)


@jax.jit
def kernel(x, h, edge_index, W_msg, W_z, b_z, W_r, b_r, W_h, b_h):
    ei = edge_index.astype(jnp.int32).reshape(2 * E)
    zeros = jnp.zeros((ROWS_PER_TILE, D), jnp.float32)
    partials = _sc_agg(x, ei, zeros)
    return _gru_tc(partials, h, W_msg, W_z, W_r, W_h, b_z, b_r, b_h)
